# Initial kernel scaffold; baseline (speedup 1.0000x reference)
import jax
import jax.numpy as jnp
from jax.experimental import pallas as pl


def kernel(xi, mu_N, h, hc, s, sc, W1, W2, W3, W4, W5, W6, W7, W8, W9, W10):
    raise NotImplementedError("write your pallas kernel here")


# speedup vs baseline: 4.9124x; 4.9124x over previous
import jax
import jax.numpy as jnp
from jax.experimental import pallas as pl
from jax.experimental.pallas import tpu as pltpu

P_DIM = 128
E = 320000
ROWS = 8000
G = E // ROWS
AUX_R = E // P_DIM


def _body(mu_ref, aux_ref, w1t_ref, w2t_ref, w4t_ref, w6t_ref, w8t_ref,
          wv_ref, w10t_ref, xi_ref, out_ref, acc_ref):
    k = pl.program_id(0)

    @pl.when(k == 0)
    def _init():
        acc_ref[...] = jnp.zeros_like(acc_ref)

    blk = mu_ref[...]
    acc_ref[...] += jnp.sum(blk.reshape(ROWS // 8, 8, P_DIM), axis=0)

    @pl.when(k == G - 1)
    def _finish():
        s = jnp.sum(acc_ref[...], axis=0, keepdims=True)
        aux = aux_ref[...]
        p = jnp.sum(jnp.maximum(aux, 0.0), axis=(1, 2), keepdims=True)[:, :, 0]
        n = jnp.sum(jnp.maximum(-aux, 0.0), axis=(1, 2), keepdims=True)[:, :, 0]
        wv = wv_ref[...]
        v = p * jnp.maximum(wv, 0.0) + n * jnp.maximum(-wv, 0.0)
        tmp = jnp.dot(s, w1t_ref[...], preferred_element_type=jnp.float32)
        tmp += jnp.dot(v[0:1], w2t_ref[...], preferred_element_type=jnp.float32)
        tmp += jnp.dot(v[1:2], w4t_ref[...], preferred_element_type=jnp.float32)
        tmp += jnp.dot(v[2:3], w6t_ref[...], preferred_element_type=jnp.float32)
        tmp += jnp.dot(v[3:4], w8t_ref[...], preferred_element_type=jnp.float32)
        tmp += jnp.dot(xi_ref[...], w10t_ref[...], preferred_element_type=jnp.float32)
        out_ref[...] = jnp.maximum(tmp, 0.0)


def kernel(xi, mu_N, h, hc, s, sc, W1, W2, W3, W4, W5, W6, W7, W8, W9, W10):
    aux = jnp.stack([h.reshape(AUX_R, P_DIM), hc.reshape(AUX_R, P_DIM),
                     s.reshape(AUX_R, P_DIM), sc.reshape(AUX_R, P_DIM)])
    wv = jnp.stack([W3[:, 0], W5[:, 0], W7[:, 0], W9[:, 0]])
    full = lambda shape: pl.BlockSpec(shape, lambda k: (0,) * len(shape))
    out = pl.pallas_call(
        _body,
        grid=(G,),
        in_specs=[
            pl.BlockSpec((ROWS, P_DIM), lambda k: (k, 0)),
            full((4, AUX_R, P_DIM)),
            full((P_DIM, P_DIM)), full((P_DIM, P_DIM)), full((P_DIM, P_DIM)),
            full((P_DIM, P_DIM)), full((P_DIM, P_DIM)),
            full((4, P_DIM)),
            full((2, P_DIM)),
            full((1, 2)),
        ],
        out_specs=full((1, P_DIM)),
        out_shape=jax.ShapeDtypeStruct((1, P_DIM), jnp.float32),
        scratch_shapes=[pltpu.VMEM((8, P_DIM), jnp.float32)],
        compiler_params=pltpu.CompilerParams(
            dimension_semantics=("arbitrary",)),
    )(mu_N, aux, W1.T, W2.T, W4.T, W6.T, W8.T, wv, W10.T,
      xi.reshape(1, 2))
    return out.reshape(P_DIM)
